# Initial kernel scaffold; baseline (speedup 1.0000x reference)
import jax
import jax.numpy as jnp
from jax.experimental import pallas as pl


def kernel(logits, labels):
    raise NotImplementedError("write your pallas kernel here")


# speedup vs baseline: 1.2488x; 1.2488x over previous
import functools

import jax
import jax.numpy as jnp
import numpy as np
from jax.experimental import pallas as pl
from jax.experimental.pallas import tpu as pltpu

N_BINS = 15
N_ROWS = 1048576
N_CLASSES = 128
ROWS_PER_BLOCK = 4096
N_BLOCKS = N_ROWS // ROWS_PER_BLOCK

_EDGES64 = np.linspace(0.0, 1.0, N_BINS + 1)
_LO = _EDGES64.astype(np.float32)
_HI = (_EDGES64 + 1.0 / N_BINS).astype(np.float32)


def _ece_kernel(logits_ref, labels_ref, edges_ref, out_ref,
                cnt_ref, sconf_ref, sacc_ref):
    i = pl.program_id(0)

    @pl.when(i == 0)
    def _init():
        cnt_ref[...] = jnp.zeros_like(cnt_ref)
        sconf_ref[...] = jnp.zeros_like(sconf_ref)
        sacc_ref[...] = jnp.zeros_like(sacc_ref)

    x = logits_ref[...]
    lbl = labels_ref[0, 0, :]

    m = jnp.max(x, axis=1, keepdims=True)
    e = jnp.exp(x - m)
    s = jax.lax.dot_general(
        e, jnp.ones((N_CLASSES, 1), jnp.float32),
        (((1,), (0,)), ((), ())),
        preferred_element_type=jnp.float32,
    )
    conf = 1.0 / s

    iota = jax.lax.broadcasted_iota(jnp.int32, x.shape, 1)
    pred = jnp.min(jnp.where(x == m, iota, N_CLASSES), axis=1)
    acc = (pred == lbl).astype(jnp.float32)

    lo = edges_ref[0:1, :]
    hi = edges_ref[1:2, :]
    inb = ((conf > lo) & (conf <= hi)).astype(jnp.float32)

    cnt_ref[...] += jnp.sum(inb, axis=0, keepdims=True)
    sconf_ref[...] += jnp.sum(inb * conf, axis=0, keepdims=True)
    sacc_ref[...] += jnp.sum(inb * acc[:, None], axis=0, keepdims=True)

    @pl.when(i == N_BLOCKS - 1)
    def _finish():
        cnt = cnt_ref[...]
        prop = cnt * (1.0 / N_ROWS)
        safe = jnp.maximum(cnt, 1.0)
        avg_conf = sconf_ref[...] / safe
        avg_acc = sacc_ref[...] / safe
        contrib = jnp.abs(avg_conf - avg_acc) * prop
        ece = jnp.sum(jnp.where(prop > 0, contrib, 0.0))
        out_ref[...] = jnp.full(out_ref.shape, ece, jnp.float32)


@jax.jit
def _ece(logits, labels):
    labels3 = labels.astype(jnp.int32).reshape(N_BLOCKS, 1, ROWS_PER_BLOCK)
    out = pl.pallas_call(
        _ece_kernel,
        grid=(N_BLOCKS,),
        in_specs=[
            pl.BlockSpec((ROWS_PER_BLOCK, N_CLASSES), lambda i: (i, 0)),
            pl.BlockSpec((1, 1, ROWS_PER_BLOCK), lambda i: (i, 0, 0)),
            pl.BlockSpec((2, N_BINS + 1), lambda i: (0, 0)),
        ],
        out_specs=pl.BlockSpec((8, 128), lambda i: (0, 0)),
        out_shape=jax.ShapeDtypeStruct((8, 128), jnp.float32),
        scratch_shapes=[
            pltpu.VMEM((1, N_BINS + 1), jnp.float32),
            pltpu.VMEM((1, N_BINS + 1), jnp.float32),
            pltpu.VMEM((1, N_BINS + 1), jnp.float32),
        ],
    )(logits, labels3, jnp.asarray(np.stack([_LO, _HI])))
    return out[0, 0].reshape(1)


def kernel(logits, labels):
    return _ece(logits, labels)


# speedup vs baseline: 2.9104x; 2.3305x over previous
import jax
import jax.numpy as jnp
import numpy as np
from jax.experimental import pallas as pl
from jax.experimental.pallas import tpu as pltpu

N_BINS = 15
N_ROWS = 1048576
N_CLASSES = 128
ROWS_PER_BLOCK = 4096
N_BLOCKS = N_ROWS // ROWS_PER_BLOCK
LANE = 128

_EDGES64 = np.linspace(0.0, 1.0, N_BINS + 1)
_LO = _EDGES64.astype(np.float32)
_HI = (_EDGES64 + 1.0 / N_BINS).astype(np.float32)


def _ece_kernel(logits_ref, labels_ref, edges_ref, out_ref,
                cnt_ref, sconf_ref, sacc_ref):
    i = pl.program_id(0)
    R = ROWS_PER_BLOCK

    @pl.when(i == 0)
    def _init():
        cnt_ref[...] = jnp.zeros_like(cnt_ref)
        sconf_ref[...] = jnp.zeros_like(sconf_ref)
        sacc_ref[...] = jnp.zeros_like(sacc_ref)

    x = logits_ref[...]
    lbl = labels_ref[0, 0, :]

    m = jnp.max(x, axis=1, keepdims=True)
    t = x - m
    e = jnp.exp(t)

    ones_row = jnp.ones((1, N_CLASSES), jnp.float32)
    sT = jax.lax.dot_general(
        ones_row, e, (((1,), (1,)), ((), ())),
        preferred_element_type=jnp.float32)
    iota = jax.lax.broadcasted_iota(jnp.int32, x.shape, 1)
    tl = jnp.where(iota == lbl[:, None], t, 0.0)
    tlT = jax.lax.dot_general(
        ones_row, tl, (((1,), (1,)), ((), ())),
        preferred_element_type=jnp.float32)

    confT = 1.0 / sT
    accT = (tlT == 0.0).astype(jnp.float32)

    lo = edges_ref[:, 0:1]
    hi = edges_ref[:, 1:2]
    inb = ((confT > lo) & (confT <= hi)).astype(jnp.float32)

    cnt_ref[...] += jnp.sum(
        inb.reshape(N_BINS + 1, R // LANE, LANE), axis=1)
    sconf_ref[...] += jnp.sum(
        (inb * confT).reshape(N_BINS + 1, R // LANE, LANE), axis=1)
    sacc_ref[...] += jnp.sum(
        (inb * accT).reshape(N_BINS + 1, R // LANE, LANE), axis=1)

    @pl.when(i == N_BLOCKS - 1)
    def _finish():
        cnt = jnp.sum(cnt_ref[...], axis=1, keepdims=True)
        sc = jnp.sum(sconf_ref[...], axis=1, keepdims=True)
        sa = jnp.sum(sacc_ref[...], axis=1, keepdims=True)
        prop = cnt * (1.0 / N_ROWS)
        safe = jnp.maximum(cnt, 1.0)
        contrib = jnp.abs(sc / safe - sa / safe) * prop
        ece = jnp.sum(jnp.where(prop > 0, contrib, 0.0))
        out_ref[...] = jnp.full(out_ref.shape, ece, jnp.float32)


@jax.jit
def _ece(logits, labels):
    labels3 = labels.astype(jnp.int32).reshape(N_BLOCKS, 1, ROWS_PER_BLOCK)
    edges = jnp.asarray(np.stack([_LO, _HI], axis=1))
    out = pl.pallas_call(
        _ece_kernel,
        grid=(N_BLOCKS,),
        in_specs=[
            pl.BlockSpec((ROWS_PER_BLOCK, N_CLASSES), lambda i: (i, 0)),
            pl.BlockSpec((1, 1, ROWS_PER_BLOCK), lambda i: (i, 0, 0)),
            pl.BlockSpec((N_BINS + 1, 2), lambda i: (0, 0)),
        ],
        out_specs=pl.BlockSpec((8, 128), lambda i: (0, 0)),
        out_shape=jax.ShapeDtypeStruct((8, 128), jnp.float32),
        scratch_shapes=[
            pltpu.VMEM((N_BINS + 1, LANE), jnp.float32),
            pltpu.VMEM((N_BINS + 1, LANE), jnp.float32),
            pltpu.VMEM((N_BINS + 1, LANE), jnp.float32),
        ],
    )(logits, labels3, edges)
    return out[0, 0].reshape(1)


def kernel(logits, labels):
    return _ece(logits, labels)


# speedup vs baseline: 3.1580x; 1.0851x over previous
import jax
import jax.numpy as jnp
import numpy as np
from jax.experimental import pallas as pl
from jax.experimental.pallas import tpu as pltpu

N_BINS = 15
N_ROWS = 1048576
N_CLASSES = 128
ROWS_PER_BLOCK = 16384
N_BLOCKS = N_ROWS // ROWS_PER_BLOCK
LANE = 128

_EDGES64 = np.linspace(0.0, 1.0, N_BINS + 1)
_LO = _EDGES64.astype(np.float32)
_HI = (_EDGES64 + 1.0 / N_BINS).astype(np.float32)


def _ece_kernel(logits_ref, labels_ref, edges_ref, out_ref,
                cnt_ref, sconf_ref, sacc_ref):
    i = pl.program_id(0)
    R = ROWS_PER_BLOCK

    @pl.when(i == 0)
    def _init():
        cnt_ref[...] = jnp.zeros_like(cnt_ref)
        sconf_ref[...] = jnp.zeros_like(sconf_ref)
        sacc_ref[...] = jnp.zeros_like(sacc_ref)

    x = logits_ref[...]
    lbl = labels_ref[0, 0, :]

    m = jnp.max(x, axis=1, keepdims=True)
    t = x - m
    e = jnp.exp(t)

    ones_row = jnp.ones((1, N_CLASSES), jnp.float32)
    sT = jax.lax.dot_general(
        ones_row, e, (((1,), (1,)), ((), ())),
        preferred_element_type=jnp.float32)
    iota = jax.lax.broadcasted_iota(jnp.int32, x.shape, 1)
    tl = jnp.where(iota == lbl[:, None], t, 0.0)
    tlT = jax.lax.dot_general(
        ones_row, tl, (((1,), (1,)), ((), ())),
        preferred_element_type=jnp.float32)

    confT = 1.0 / sT
    accT = (tlT == 0.0).astype(jnp.float32)

    lo = edges_ref[:, 0:1]
    hi = edges_ref[:, 1:2]
    inb = ((confT > lo) & (confT <= hi)).astype(jnp.float32)

    cnt_ref[...] += jnp.sum(
        inb.reshape(N_BINS + 1, R // LANE, LANE), axis=1)
    sconf_ref[...] += jnp.sum(
        (inb * confT).reshape(N_BINS + 1, R // LANE, LANE), axis=1)
    sacc_ref[...] += jnp.sum(
        (inb * accT).reshape(N_BINS + 1, R // LANE, LANE), axis=1)

    @pl.when(i == N_BLOCKS - 1)
    def _finish():
        cnt = jnp.sum(cnt_ref[...], axis=1, keepdims=True)
        sc = jnp.sum(sconf_ref[...], axis=1, keepdims=True)
        sa = jnp.sum(sacc_ref[...], axis=1, keepdims=True)
        prop = cnt * (1.0 / N_ROWS)
        safe = jnp.maximum(cnt, 1.0)
        contrib = jnp.abs(sc / safe - sa / safe) * prop
        ece = jnp.sum(jnp.where(prop > 0, contrib, 0.0))
        out_ref[...] = jnp.full(out_ref.shape, ece, jnp.float32)


@jax.jit
def _ece(logits, labels):
    labels3 = labels.astype(jnp.int32).reshape(N_BLOCKS, 1, ROWS_PER_BLOCK)
    edges = jnp.asarray(np.stack([_LO, _HI], axis=1))
    out = pl.pallas_call(
        _ece_kernel,
        grid=(N_BLOCKS,),
        in_specs=[
            pl.BlockSpec((ROWS_PER_BLOCK, N_CLASSES), lambda i: (i, 0)),
            pl.BlockSpec((1, 1, ROWS_PER_BLOCK), lambda i: (i, 0, 0)),
            pl.BlockSpec((N_BINS + 1, 2), lambda i: (0, 0)),
        ],
        out_specs=pl.BlockSpec((8, 128), lambda i: (0, 0)),
        out_shape=jax.ShapeDtypeStruct((8, 128), jnp.float32),
        scratch_shapes=[
            pltpu.VMEM((N_BINS + 1, LANE), jnp.float32),
            pltpu.VMEM((N_BINS + 1, LANE), jnp.float32),
            pltpu.VMEM((N_BINS + 1, LANE), jnp.float32),
        ],
    )(logits, labels3, edges)
    return out[0, 0].reshape(1)


def kernel(logits, labels):
    return _ece(logits, labels)


# speedup vs baseline: 3.8835x; 1.2297x over previous
import jax
import jax.numpy as jnp
import numpy as np
from jax.experimental import pallas as pl
from jax.experimental.pallas import tpu as pltpu

N_BINS = 15
N_ROWS = 1048576
N_CLASSES = 128
ROWS_PER_BLOCK = 16384
N_BLOCKS = N_ROWS // ROWS_PER_BLOCK
LANE = 128

_EDGES64 = np.linspace(0.0, 1.0, N_BINS + 1)
_LO = _EDGES64.astype(np.float32)


def _ece_kernel(logits_ref, labels_ref, edges_ref, out_ref,
                cnt_ref, sconf_ref, sacc_ref):
    i = pl.program_id(0)
    R = ROWS_PER_BLOCK

    @pl.when(i == 0)
    def _init():
        cnt_ref[...] = jnp.zeros_like(cnt_ref)
        sconf_ref[...] = jnp.zeros_like(sconf_ref)
        sacc_ref[...] = jnp.zeros_like(sacc_ref)

    x = logits_ref[...]
    lbl = labels_ref[0, 0, :]

    m = jnp.max(x, axis=1, keepdims=True)
    t = x - m
    e = jnp.exp(t)

    ones_row = jnp.ones((1, N_CLASSES), jnp.float32)
    sT = jax.lax.dot_general(
        ones_row, e, (((1,), (1,)), ((), ())),
        preferred_element_type=jnp.float32)
    iota = jax.lax.broadcasted_iota(jnp.int16, x.shape, 1)
    tb = t.astype(jnp.bfloat16)
    tl = jnp.where(iota == lbl[:, None], tb, jnp.bfloat16(0))
    tlT = jax.lax.dot_general(
        jnp.ones((1, N_CLASSES), jnp.bfloat16), tl,
        (((1,), (1,)), ((), ())),
        preferred_element_type=jnp.float32)

    confT = 1.0 / sT
    accT = (tlT == 0.0).astype(jnp.float32)

    lo = edges_ref[:, 0:1]
    g = confT > lo
    gf = g.astype(jnp.float32)

    cnt_ref[...] += jnp.sum(
        gf.reshape(N_BINS + 1, R // LANE, LANE), axis=1)
    sconf_ref[...] += jnp.sum(
        jnp.where(g, confT, 0.0).reshape(N_BINS + 1, R // LANE, LANE), axis=1)
    sacc_ref[...] += jnp.sum(
        jnp.where(g, accT, 0.0).reshape(N_BINS + 1, R // LANE, LANE), axis=1)

    @pl.when(i == N_BLOCKS - 1)
    def _finish():
        gcnt = jnp.sum(cnt_ref[...], axis=1, keepdims=True)
        gsc = jnp.sum(sconf_ref[...], axis=1, keepdims=True)
        gsa = jnp.sum(sacc_ref[...], axis=1, keepdims=True)
        zero = jnp.zeros((1, 1), jnp.float32)
        cnt = gcnt - jnp.concatenate([gcnt[1:], zero], axis=0)
        sc = gsc - jnp.concatenate([gsc[1:], zero], axis=0)
        sa = gsa - jnp.concatenate([gsa[1:], zero], axis=0)
        prop = cnt * (1.0 / N_ROWS)
        safe = jnp.maximum(cnt, 1.0)
        contrib = jnp.abs(sc / safe - sa / safe) * prop
        ece = jnp.sum(jnp.where(prop > 0, contrib, 0.0))
        out_ref[...] = jnp.full(out_ref.shape, ece, jnp.float32)


@jax.jit
def _ece(logits, labels):
    labels3 = labels.astype(jnp.int16).reshape(N_BLOCKS, 1, ROWS_PER_BLOCK)
    edges = jnp.asarray(np.stack([_LO, _LO], axis=1))
    out = pl.pallas_call(
        _ece_kernel,
        grid=(N_BLOCKS,),
        in_specs=[
            pl.BlockSpec((ROWS_PER_BLOCK, N_CLASSES), lambda i: (i, 0)),
            pl.BlockSpec((1, 1, ROWS_PER_BLOCK), lambda i: (i, 0, 0)),
            pl.BlockSpec((N_BINS + 1, 2), lambda i: (0, 0)),
        ],
        out_specs=pl.BlockSpec((8, 128), lambda i: (0, 0)),
        out_shape=jax.ShapeDtypeStruct((8, 128), jnp.float32),
        scratch_shapes=[
            pltpu.VMEM((N_BINS + 1, LANE), jnp.float32),
            pltpu.VMEM((N_BINS + 1, LANE), jnp.float32),
            pltpu.VMEM((N_BINS + 1, LANE), jnp.float32),
        ],
    )(logits, labels3, edges)
    return out[0, 0].reshape(1)


def kernel(logits, labels):
    return _ece(logits, labels)
